# Initial kernel scaffold; baseline (speedup 1.0000x reference)
import jax
import jax.numpy as jnp
from jax.experimental import pallas as pl


def kernel(cosine, label):
    raise NotImplementedError("write your pallas kernel here")


# speedup vs baseline: 1.3621x; 1.3621x over previous
import functools
import math

import jax
import jax.numpy as jnp
from jax import lax
from jax.experimental import pallas as pl
from jax.experimental.pallas import tpu as pltpu
from jax.experimental.pallas import tpu_sc as plsc

_B, _C = 1024, 100000
_S = 64.0
_M2 = 0.3
_M3 = 0.2
_CM2 = math.cos(_M2)
_SM2 = math.sin(_M2)

_NC, _NS, _L = 2, 16, 16
_NW = _NC * _NS
_RPW = _B // _NW


def _sc_margin_body(flat_hbm, label_hbm, v_hbm, lab_v, idx_v, x_v, out_v, sem):
    wid = lax.axis_index("s") * _NC + lax.axis_index("c")
    base = wid * _RPW
    pltpu.sync_copy(label_hbm.at[pl.ds(base, _RPW)], lab_v)
    for k in range(_RPW // _L):
        lab16 = jnp.maximum(lab_v[pl.ds(k * _L, _L)], 0)
        rows16 = (base + k * _L) + lax.iota(jnp.int32, _L)
        idx_v[pl.ds(k * _L, _L)] = rows16 * _C + lab16
    pltpu.async_copy(flat_hbm.at[idx_v], x_v, sem).wait()
    for k in range(_RPW // _L):
        x = x_v[pl.ds(k * _L, _L)]
        y = jnp.maximum(1.0 - x * x, 1e-12)
        i = lax.bitcast_convert_type(y, jnp.int32)
        r = lax.bitcast_convert_type(0x5F3759DF - (i >> 1), jnp.float32)
        for _ in range(3):
            r = r * (1.5 - 0.5 * y * r * r)
        sq = y * r
        out_v[pl.ds(k * _L, _L)] = (x * _CM2 - sq * _SM2 - _M3) * _S
    pltpu.sync_copy(out_v, v_hbm.at[pl.ds(base, _RPW)])


@functools.cache
def _sc_margin():
    return pl.kernel(
        _sc_margin_body,
        mesh=plsc.VectorSubcoreMesh(core_axis_name="c", subcore_axis_name="s"),
        out_type=jax.ShapeDtypeStruct((_B,), jnp.float32),
        scratch_types=[
            pltpu.VMEM((_RPW,), jnp.int32),
            pltpu.VMEM((_RPW,), jnp.int32),
            pltpu.VMEM((_RPW,), jnp.float32),
            pltpu.VMEM((_RPW,), jnp.float32),
            pltpu.SemaphoreType.DMA,
        ],
    )


def _tc_body(x_ref, lab_ref, v_ref, o_ref):
    x = x_ref[...]
    cols = lax.broadcasted_iota(jnp.int32, x.shape, 1)
    mask = cols == lab_ref[...]
    o_ref[...] = jnp.where(mask, v_ref[...], x * _S)


def _tc_stream(cosine, lab2, v2, bm):
    return pl.pallas_call(
        _tc_body,
        grid=(_B // bm,),
        in_specs=[
            pl.BlockSpec((bm, _C), lambda i: (i, 0)),
            pl.BlockSpec((bm, 1), lambda i: (i, 0)),
            pl.BlockSpec((bm, 1), lambda i: (i, 0)),
        ],
        out_specs=pl.BlockSpec((bm, _C), lambda i: (i, 0)),
        out_shape=jax.ShapeDtypeStruct((_B, _C), jnp.float32),
    )(cosine, lab2, v2)


def kernel(cosine, label):
    v = _sc_margin()(cosine.reshape(_B * _C), label)
    return _tc_stream(cosine, label.reshape(_B, 1), v.reshape(_B, 1), 8)


# speedup vs baseline: 1.3696x; 1.0055x over previous
import functools
import math

import jax
import jax.numpy as jnp
from jax import lax
from jax.experimental import pallas as pl
from jax.experimental.pallas import tpu as pltpu
from jax.experimental.pallas import tpu_sc as plsc

_B, _C = 1024, 100000
_S = 64.0
_M2 = 0.3
_M3 = 0.2
_CM2 = math.cos(_M2)
_SM2 = math.sin(_M2)

_NC, _NS, _L = 2, 16, 16
_NW = _NC * _NS
_RPW = _B // _NW


def _sc_margin_body(flat_hbm, label_hbm, v_hbm, lab_v, idx_v, x_v, out_v, sem):
    wid = lax.axis_index("s") * _NC + lax.axis_index("c")
    base = wid * _RPW
    pltpu.sync_copy(label_hbm.at[pl.ds(base, _RPW)], lab_v)
    for k in range(_RPW // _L):
        lab16 = jnp.maximum(lab_v[pl.ds(k * _L, _L)], 0)
        rows16 = (base + k * _L) + lax.iota(jnp.int32, _L)
        idx_v[pl.ds(k * _L, _L)] = rows16 * _C + lab16
    pltpu.async_copy(flat_hbm.at[idx_v], x_v, sem).wait()
    for k in range(_RPW // _L):
        x = x_v[pl.ds(k * _L, _L)]
        y = jnp.maximum(1.0 - x * x, 1e-12)
        i = lax.bitcast_convert_type(y, jnp.int32)
        r = lax.bitcast_convert_type(0x5F3759DF - (i >> 1), jnp.float32)
        for _ in range(3):
            r = r * (1.5 - 0.5 * y * r * r)
        sq = y * r
        out_v[pl.ds(k * _L, _L)] = (x * _CM2 - sq * _SM2 - _M3) * _S
    pltpu.sync_copy(out_v, v_hbm.at[pl.ds(base, _RPW)])


@functools.cache
def _sc_margin():
    return pl.kernel(
        _sc_margin_body,
        mesh=plsc.VectorSubcoreMesh(core_axis_name="c", subcore_axis_name="s"),
        out_type=jax.ShapeDtypeStruct((_B,), jnp.float32),
        scratch_types=[
            pltpu.VMEM((_RPW,), jnp.int32),
            pltpu.VMEM((_RPW,), jnp.int32),
            pltpu.VMEM((_RPW,), jnp.float32),
            pltpu.VMEM((_RPW,), jnp.float32),
            pltpu.SemaphoreType.DMA,
        ],
    )


def _tc_body(x_ref, lab_ref, v_ref, o_ref):
    x = x_ref[...]
    cols = lax.broadcasted_iota(jnp.int32, x.shape, 1)
    mask = cols == lab_ref[...]
    o_ref[...] = jnp.where(mask, v_ref[...], x * _S)


def _tc_stream(cosine, lab2, v2, bm):
    return pl.pallas_call(
        _tc_body,
        grid=(_B // bm,),
        in_specs=[
            pl.BlockSpec((bm, _C), lambda i: (i, 0)),
            pl.BlockSpec((bm, 1), lambda i: (i, 0)),
            pl.BlockSpec((bm, 1), lambda i: (i, 0)),
        ],
        out_specs=pl.BlockSpec((bm, _C), lambda i: (i, 0)),
        out_shape=jax.ShapeDtypeStruct((_B, _C), jnp.float32),
    )(cosine, lab2, v2)


def kernel(cosine, label):
    v = _sc_margin()(cosine.reshape(_B * _C), label)
    return _tc_stream(cosine, label.reshape(_B, 1), v.reshape(_B, 1), 16)
